# baseline (speedup 1.0000x reference)
import functools

import jax
import jax.numpy as jnp
from jax import lax
from jax.experimental import pallas as pl
from jax.experimental.pallas import tpu as pltpu
from jax.experimental.pallas import tpu_sc as plsc

BATCH = 16384
NF = 16
NW = 32
BPW = BATCH // NW
CHB = 128
NCH = BPW // CHB

_mesh = plsc.VectorSubcoreMesh(core_axis_name="c", subcore_axis_name="s")


@functools.partial(
    pl.kernel,
    mesh=_mesh,
    compiler_params=pltpu.CompilerParams(
        needs_layout_passes=False, use_tc_tiling_on_sc=False),
    out_type=jax.ShapeDtypeStruct((BATCH,), jnp.float32),
    scratch_types=[
        pltpu.VMEM((NCH, CHB), jnp.int32),
        pltpu.VMEM((NCH, CHB), jnp.int32),
        pltpu.VMEM((BPW, NF), jnp.float32),
        pltpu.VMEM((BPW, NF), jnp.float32),
        pltpu.VMEM((BPW,), jnp.float32),
        pltpu.SemaphoreType.DMA,
        pltpu.SemaphoreType.DMA,
    ],
)
def _mf_kernel(u_hbm, i_hbm, ut_hbm, bt_hbm, out_hbm,
               uidx, iidx, urows, brows, outv, usem, bsem):
    wid = lax.axis_index("s") * 2 + lax.axis_index("c")
    base = wid * BPW

    pltpu.sync_copy(u_hbm.at[pl.ds(wid * NCH, NCH)], uidx)
    pltpu.sync_copy(i_hbm.at[pl.ds(wid * NCH, NCH)], iidx)

    copies = []
    for c in range(NCH):
        copies.append(pltpu.async_copy(
            ut_hbm.at[uidx.at[c]], urows.at[pl.ds(c * CHB, CHB)], usem))
        copies.append(pltpu.async_copy(
            bt_hbm.at[iidx.at[c]], brows.at[pl.ds(c * CHB, CHB)], bsem))
    for cp in copies:
        cp.wait()

    lanes = lax.iota(jnp.int32, NF)

    def body(g, carry):
        gb = g * NF
        rows = gb + lanes
        acc = jnp.zeros((NF,), jnp.float32)
        for f in range(NF):
            cols = jnp.full((NF,), f, jnp.int32)
            uv = plsc.load_gather(urows, [rows, cols])
            bv = plsc.load_gather(brows, [rows, cols])
            acc = acc + uv * bv
        outv[pl.ds(gb, NF)] = 1.0 / (1.0 + jnp.exp(-acc))
        return carry

    lax.fori_loop(0, BPW // NF, body, 0)

    pltpu.sync_copy(outv, out_hbm.at[pl.ds(base, BPW)])


def kernel(u, i, user_table, book_table):
    u2 = jnp.reshape(u.astype(jnp.int32), (NW * NCH, CHB))
    i2 = jnp.reshape(i.astype(jnp.int32), (NW * NCH, CHB))
    return _mf_kernel(u2, i2, user_table, book_table)


# speedup vs baseline: 6.6569x; 6.6569x over previous
import functools

import jax
import jax.numpy as jnp
from jax import lax
from jax.experimental import pallas as pl
from jax.experimental.pallas import tpu as pltpu
from jax.experimental.pallas import tpu_sc as plsc

BATCH = 16384
NF = 16
NROWS = 1000001
PLANE_W = 1000064
PART = 62464
REM = 512
TAIL_BASE = 999936

_mesh = plsc.VectorSubcoreMesh(core_axis_name="c", subcore_axis_name="s")
_params = pltpu.CompilerParams(
    needs_layout_passes=False, use_tc_tiling_on_sc=True)


@functools.partial(
    pl.kernel,
    mesh=_mesh,
    compiler_params=_params,
    out_type=jax.ShapeDtypeStruct((2, 128, 128), jnp.float32),
    scratch_types=[
        pltpu.VMEM((8, 128), jnp.int32),
        pltpu.VMEM((8, 128), jnp.int32),
        pltpu.VMEM((8, 128), jnp.float32),
        pltpu.VMEM((8, 128), jnp.float32),
        pltpu.VMEM((8, 128), jnp.float32),
        pltpu.VMEM_SHARED((PLANE_W,), jnp.float32),
        pltpu.VMEM_SHARED((PLANE_W,), jnp.float32),
        pltpu.SemaphoreType.DMA,
        pltpu.SemaphoreType.DMA,
    ],
)
def _mf_partial(u2, i2, ut3, bt3, tu3, tb3, part_out,
                uidx, iidx, gu, gb, acc, ring0, ring1, ssem, gsem):
    c = lax.axis_index("c")
    s = lax.axis_index("s")
    row8 = pl.multiple_of(s * 8, 8)
    part_off = pl.multiple_of(s * PART, 128)

    pltpu.sync_copy(u2.at[pl.ds(row8, 8)], uidx)
    pltpu.sync_copy(i2.at[pl.ds(row8, 8)], iidx)

    def plane_src(p):
        tab = ut3 if p % 2 == 0 else bt3
        return tab.at[c].at[p // 2]

    def tail_src(p):
        tab = tu3 if p % 2 == 0 else tb3
        return tab.at[c].at[p // 2]

    def stream_copies(p):
        src = plane_src(p)
        slot = ring0 if p % 2 == 0 else ring1
        yield src.at[pl.ds(part_off, PART)], slot.at[pl.ds(part_off, PART)]

    def stream_extra(p):
        src = plane_src(p)
        slot = ring0 if p % 2 == 0 else ring1
        yield (src.at[pl.ds(16 * PART, REM)],
               slot.at[pl.ds(16 * PART, REM)])
        yield (tail_src(p), slot.at[pl.ds(TAIL_BASE, 128)])

    def issue_stream(p):
        for a, b in stream_copies(p):
            pltpu.async_copy(a, b, ssem)

        @pl.when(s == 0)
        def _():
            for a, b in stream_extra(p):
                pltpu.async_copy(a, b, ssem)

    def wait_stream(p):
        for a, b in stream_copies(p):
            pltpu.make_async_copy(a, b, ssem).wait()

        @pl.when(s == 0)
        def _():
            for a, b in stream_extra(p):
                pltpu.make_async_copy(a, b, ssem).wait()

    issue_stream(0)
    for p in range(2 * 8):
        wait_stream(p)
        plsc.subcore_barrier()
        if p + 1 < 2 * 8:
            issue_stream(p + 1)
        slot = ring0 if p % 2 == 0 else ring1
        idx = uidx if p % 2 == 0 else iidx
        dst = gu if p % 2 == 0 else gb
        copies = [
            pltpu.async_copy(slot.at[idx.at[j]], dst.at[j], gsem)
            for j in range(8)
        ]
        for cp in copies:
            cp.wait()
        if p % 2 == 1:
            first = p == 1

            def fma(t, carry):
                j = t // 8
                o = pl.multiple_of((t % 8) * 16, 16)
                prod = gu[j, pl.ds(o, 16)] * gb[j, pl.ds(o, 16)]
                if first:
                    acc[j, pl.ds(o, 16)] = prod
                else:
                    acc[j, pl.ds(o, 16)] = acc[j, pl.ds(o, 16)] + prod
                return carry

            lax.fori_loop(0, 64, fma, 0, unroll=8)

    pltpu.sync_copy(acc, part_out.at[c].at[pl.ds(row8, 8)])


@functools.partial(
    pl.kernel,
    mesh=_mesh,
    compiler_params=_params,
    out_type=jax.ShapeDtypeStruct((128, 128), jnp.float32),
    scratch_types=[
        pltpu.VMEM((8, 128), jnp.float32),
        pltpu.VMEM((8, 128), jnp.float32),
        pltpu.VMEM((8, 128), jnp.float32),
    ],
)
def _mf_combine(part, out_hbm, p0, p1, o):
    c = lax.axis_index("c")
    s = lax.axis_index("s")
    row8 = pl.multiple_of(s * 8, 8)

    @pl.when(c == 0)
    def _():
        pltpu.sync_copy(part.at[0].at[pl.ds(row8, 8)], p0)
        pltpu.sync_copy(part.at[1].at[pl.ds(row8, 8)], p1)

        def body(t, carry):
            j = t // 8
            off = pl.multiple_of((t % 8) * 16, 16)
            x = p0[j, pl.ds(off, 16)] + p1[j, pl.ds(off, 16)]
            o[j, pl.ds(off, 16)] = 1.0 / (1.0 + jnp.exp(-x))
            return carry

        lax.fori_loop(0, 64, body, 0, unroll=8)
        pltpu.sync_copy(o, out_hbm.at[pl.ds(row8, 8)])


def _tail_planes(table):
    t = jnp.pad(table[TAIL_BASE:1000000], ((0, 64), (0, 0)))
    return jnp.reshape(t.T, (2, 8, 128))


def kernel(u, i, user_table, book_table):
    u2 = jnp.reshape(u.astype(jnp.int32), (128, 128))
    i2 = jnp.reshape(i.astype(jnp.int32), (128, 128))
    ut3 = jnp.reshape(user_table.T, (2, 8, NROWS))
    bt3 = jnp.reshape(book_table.T, (2, 8, NROWS))
    parts = _mf_partial(u2, i2, ut3, bt3,
                        _tail_planes(user_table), _tail_planes(book_table))
    out2 = _mf_combine(parts)
    return jnp.reshape(out2, (BATCH,))
